# baseline (speedup 1.0000x reference)
import functools

import numpy as _np

import jax
import jax.numpy as jnp
from jax import lax
from jax.experimental import pallas as pl
from jax.experimental.pallas import tpu as pltpu
from jax.experimental.pallas import tpu_sc as plsc

VOCAB = 100000
EMBED_DIM = 64
BATCH = 4096
SEQ_LEN = 200

NC = 2
NS = 16
NW = NC * NS

BBLK = BATCH // NW
DT = EMBED_DIM // 8
K = 2
GROUPS = SEQ_LEN // K


@functools.partial(
    pl.kernel,
    out_type=jax.ShapeDtypeStruct((SEQ_LEN, DT, NW, 8, 128), jnp.float32),
    mesh=plsc.VectorSubcoreMesh(core_axis_name="c", subcore_axis_name="s"),
    compiler_params=pltpu.CompilerParams(use_tc_tiling_on_sc=False,
                                         needs_layout_passes=False),
    scratch_types=[
        pltpu.VMEM((SEQ_LEN, BBLK), jnp.int32),
        pltpu.VMEM((2, K, BBLK, EMBED_DIM), jnp.float32),
        pltpu.VMEM((2, K, DT, 8, 128), jnp.float32),
        pltpu.SemaphoreType.DMA,
        pltpu.SemaphoreType.DMA,
        pltpu.SemaphoreType.DMA,
    ],
)
def _embed_lookup(idx_hbm, table_hbm, out_hbm, idx_v, rows_v, tile_v, gsem,
                  ssem0, ssem1):
    ssem = (ssem0, ssem1)
    wid = lax.axis_index("s") * NC + lax.axis_index("c")
    pltpu.sync_copy(idx_hbm.at[:, pl.ds(wid * BBLK, BBLK)], idx_v)

    def gather_fire(g, h):
        for b in range(K):
            pltpu.async_copy(table_hbm.at[idx_v.at[g * K + b]],
                             rows_v.at[h, b], gsem)

    def gather_drain(g, h):
        for b in range(K):
            pltpu.make_async_copy(table_hbm.at[idx_v.at[g * K + b]],
                                  rows_v.at[h, b], gsem).wait()

    def store_fire(g, h):
        for b in range(K):
            pltpu.async_copy(tile_v.at[h, b],
                             out_hbm.at[g * K + b, slice(None), wid],
                             ssem[h])

    def store_drain(g, h):
        for b in range(K):
            pltpu.make_async_copy(tile_v.at[h, b],
                                  out_hbm.at[g * K + b, slice(None), wid],
                                  ssem[h]).wait()

    lanes = lax.iota(jnp.int32, 16)
    row_idx = [lanes + 16 * k for k in range(8)]
    zeros = lanes - lanes
    r_vec = [zeros + r for r in range(8)]
    eight = zeros + 8

    def transpose(h):
        for b in range(K):
            src = rows_v.at[h, b]
            dst = tile_v.at[h, b]

            def td(dt, col):
                for r in range(8):
                    col_r = col + r_vec[r]
                    for k in range(8):
                        v = plsc.load_gather(src, [row_idx[k], col_r])
                        dst[dt, r, pl.ds(16 * k, 16)] = v
                return col + eight

            lax.fori_loop(0, DT, td, zeros)

    gather_fire(0, 0)

    def group_pair(p, carry):
        for h in range(2):
            g = 2 * p + h
            gather_drain(g, h)
            gather_fire(g + 1, 1 - h)
            store_drain(g - 2, h)
            transpose(h)
            store_fire(g, h)
        return carry

    for h in range(2):
        g = h
        gather_drain(g, h)
        gather_fire(g + 1, 1 - h)
        transpose(h)
        store_fire(g, h)

    lax.fori_loop(1, GROUPS // 2 - 1, group_pair, 0)

    for h in range(2):
        g = GROUPS - 2 + h
        gather_drain(g, h)
        if g + 1 < GROUPS:
            gather_fire(g + 1, 1 - h)
        store_drain(g - 2, h)
        transpose(h)
        store_fire(g, h)
    store_drain(GROUPS - 2, 0)
    store_drain(GROUPS - 1, 1)


def kernel(x, word2vec_matrix):
    o = _embed_lookup(x.T, word2vec_matrix)
    return o.transpose(2, 4, 0, 1, 3).reshape(BATCH, SEQ_LEN, EMBED_DIM)


# speedup vs baseline: 1.2704x; 1.2704x over previous
import functools

import numpy as _np

import jax
import jax.numpy as jnp
from jax import lax
from jax.experimental import pallas as pl
from jax.experimental.pallas import tpu as pltpu
from jax.experimental.pallas import tpu_sc as plsc

VOCAB = 100000
EMBED_DIM = 64
BATCH = 4096
SEQ_LEN = 200

NC = 2
NS = 16
NW = NC * NS

BBLK = BATCH // NW
DT = EMBED_DIM // 8
K = 2
GROUPS = SEQ_LEN // K


@functools.partial(
    pl.kernel,
    out_type=jax.ShapeDtypeStruct((SEQ_LEN, DT, NW, 8, 128), jnp.float32),
    mesh=plsc.VectorSubcoreMesh(core_axis_name="c", subcore_axis_name="s"),
    compiler_params=pltpu.CompilerParams(use_tc_tiling_on_sc=False,
                                         needs_layout_passes=False),
    scratch_types=[
        pltpu.VMEM((SEQ_LEN, BBLK), jnp.int32),
        pltpu.VMEM((2, K, BBLK, EMBED_DIM), jnp.float32),
        pltpu.VMEM((2, K, DT, 8, 128), jnp.float32),
        pltpu.SemaphoreType.DMA,
        pltpu.SemaphoreType.DMA,
        pltpu.SemaphoreType.DMA,
    ],
)
def _embed_lookup(idx_hbm, table_hbm, out_hbm, idx_v, rows_v, tile_v, gsem,
                  ssem0, ssem1):
    ssem = (ssem0, ssem1)
    wid = lax.axis_index("s") * NC + lax.axis_index("c")
    pltpu.sync_copy(idx_hbm.at[:, pl.ds(wid * BBLK, BBLK)], idx_v)

    def gather_fire(g, h):
        for b in range(K):
            pltpu.async_copy(table_hbm.at[idx_v.at[g * K + b]],
                             rows_v.at[h, b], gsem)

    def gather_drain(g, h):
        for b in range(K):
            pltpu.make_async_copy(table_hbm.at[idx_v.at[g * K + b]],
                                  rows_v.at[h, b], gsem).wait()

    def store_fire(g, h):
        for b in range(K):
            pltpu.async_copy(tile_v.at[h, b],
                             out_hbm.at[g * K + b, slice(None), wid],
                             ssem[h])

    def store_drain(g, h):
        for b in range(K):
            pltpu.make_async_copy(tile_v.at[h, b],
                                  out_hbm.at[g * K + b, slice(None), wid],
                                  ssem[h]).wait()

    lanes = lax.iota(jnp.int32, 16)
    zeros = lanes - lanes
    row_idx = [lanes + 16 * k for k in range(8)]
    r_vec = [zeros + r for r in range(8)]
    eight = zeros + 8

    def transpose(h):
        for b in range(K):
            src = rows_v.at[h, b]
            dst = tile_v.at[h, b]

            def td(dt, col):
                dst_t = dst.at[dt]
                for r in range(8):
                    col_r = col + r_vec[r]
                    vs = [plsc.load_gather(src, [row_idx[k], col_r])
                          for k in range(8)]
                    for k in range(8):
                        dst_t[r, pl.ds(16 * k, 16)] = vs[k]
                return col + eight

            lax.fori_loop(0, DT, td, zeros)

    gather_fire(0, 0)

    def group_pair(p, carry):
        for h in range(2):
            g = 2 * p + h
            gather_drain(g, h)
            gather_fire(g + 1, 1 - h)
            store_drain(g - 2, h)
            transpose(h)
            store_fire(g, h)
        return carry

    for h in range(2):
        g = h
        gather_drain(g, h)
        gather_fire(g + 1, 1 - h)
        transpose(h)
        store_fire(g, h)

    lax.fori_loop(1, GROUPS // 2 - 1, group_pair, 0)

    for h in range(2):
        g = GROUPS - 2 + h
        gather_drain(g, h)
        if g + 1 < GROUPS:
            gather_fire(g + 1, 1 - h)
        store_drain(g - 2, h)
        transpose(h)
        store_fire(g, h)
    store_drain(GROUPS - 2, 0)
    store_drain(GROUPS - 1, 1)


def kernel(x, word2vec_matrix):
    o = _embed_lookup(x.T, word2vec_matrix)
    return o.transpose(2, 4, 0, 1, 3).reshape(BATCH, SEQ_LEN, EMBED_DIM)


# speedup vs baseline: 1.2749x; 1.0036x over previous
import functools

import numpy as _np

import jax
import jax.numpy as jnp
from jax import lax
from jax.experimental import pallas as pl
from jax.experimental.pallas import tpu as pltpu
from jax.experimental.pallas import tpu_sc as plsc

VOCAB = 100000
EMBED_DIM = 64
BATCH = 4096
SEQ_LEN = 200

NC = 2
NS = 16
NW = NC * NS

BBLK = BATCH // NW
DT = EMBED_DIM // 8
K = 2
GROUPS = SEQ_LEN // K


@functools.partial(
    pl.kernel,
    out_type=jax.ShapeDtypeStruct((SEQ_LEN, DT, NW, 8, 128), jnp.float32),
    mesh=plsc.VectorSubcoreMesh(core_axis_name="c", subcore_axis_name="s"),
    compiler_params=pltpu.CompilerParams(use_tc_tiling_on_sc=False,
                                         needs_layout_passes=False),
    scratch_types=[
        pltpu.VMEM((SEQ_LEN, BBLK), jnp.int32),
        pltpu.VMEM((2, K, BBLK, EMBED_DIM), jnp.float32),
        pltpu.VMEM((2, K, DT, 8, 128), jnp.float32),
        pltpu.SemaphoreType.DMA,
        pltpu.SemaphoreType.DMA,
        pltpu.SemaphoreType.DMA,
    ],
)
def _embed_lookup(idx_hbm, table_hbm, out_hbm, idx_v, rows_v, tile_v, gsem,
                  ssem0, ssem1):
    ssem = (ssem0, ssem1)
    wid = lax.axis_index("s") * NC + lax.axis_index("c")
    pltpu.sync_copy(idx_hbm.at[:, pl.ds(wid * BBLK, BBLK)], idx_v)

    def gather_fire(g, h):
        for b in range(K):
            pltpu.async_copy(table_hbm.at[idx_v.at[g * K + b]],
                             rows_v.at[h, b], gsem)

    def gather_drain(g, h):
        for b in range(K):
            pltpu.make_async_copy(table_hbm.at[idx_v.at[g * K + b]],
                                  rows_v.at[h, b], gsem).wait()

    def store_fire(g, h):
        for b in range(K):
            pltpu.async_copy(tile_v.at[h, b],
                             out_hbm.at[g * K + b, slice(None), wid],
                             ssem[h])

    def store_drain(g, h):
        for b in range(K):
            pltpu.make_async_copy(tile_v.at[h, b],
                                  out_hbm.at[g * K + b, slice(None), wid],
                                  ssem[h]).wait()

    lanes = lax.iota(jnp.int32, 16)
    zeros = lanes - lanes
    ones = zeros + 1
    dt_vec = [lax.div(lanes + 16 * j, 8) for j in range(4)]
    dr_vec = [lax.rem(lanes + 16 * j, 8) for j in range(4)]

    def transpose(h):
        for b in range(K):
            src = rows_v.at[h, b]
            dst = tile_v.at[h, b]

            def tc(c8, cvec):
                for i in range(8):
                    c = c8 * 8 + i
                    cv = cvec + (zeros + i)
                    for j in range(4):
                        v = src[c, pl.ds(16 * j, 16)]
                        plsc.store_scatter(dst, [dt_vec[j], dr_vec[j], cv],
                                           v)
                return cvec + (zeros + 8)

            lax.fori_loop(0, BBLK // 8, tc, zeros)

    gather_fire(0, 0)

    def group_pair(p, carry):
        for h in range(2):
            g = 2 * p + h
            gather_drain(g, h)
            gather_fire(g + 1, 1 - h)
            store_drain(g - 2, h)
            transpose(h)
            store_fire(g, h)
        return carry

    for h in range(2):
        g = h
        gather_drain(g, h)
        gather_fire(g + 1, 1 - h)
        transpose(h)
        store_fire(g, h)

    lax.fori_loop(1, GROUPS // 2 - 1, group_pair, 0)

    for h in range(2):
        g = GROUPS - 2 + h
        gather_drain(g, h)
        if g + 1 < GROUPS:
            gather_fire(g + 1, 1 - h)
        store_drain(g - 2, h)
        transpose(h)
        store_fire(g, h)
    store_drain(GROUPS - 2, 0)
    store_drain(GROUPS - 1, 1)


def kernel(x, word2vec_matrix):
    o = _embed_lookup(x.T, word2vec_matrix)
    return o.transpose(2, 4, 0, 1, 3).reshape(BATCH, SEQ_LEN, EMBED_DIM)


# speedup vs baseline: 1.9196x; 1.5056x over previous
import functools

import jax
import jax.numpy as jnp
from jax import lax
from jax.experimental import pallas as pl
from jax.experimental.pallas import tpu as pltpu
from jax.experimental.pallas import tpu_sc as plsc

VOCAB = 100000
EMBED_DIM = 64
BATCH = 4096
SEQ_LEN = 200

NC = 2
NS = 16
NW = NC * NS

B_PER_W = BATCH // NW
SPLIT = 96
K = 4
GROUPS = B_PER_W // K


@functools.partial(
    pl.kernel,
    out_type=jax.ShapeDtypeStruct((BATCH, SEQ_LEN, EMBED_DIM), jnp.float32),
    mesh=plsc.VectorSubcoreMesh(core_axis_name="c", subcore_axis_name="s"),
    compiler_params=pltpu.CompilerParams(use_tc_tiling_on_sc=False),
    scratch_types=[
        pltpu.VMEM((B_PER_W, SEQ_LEN), jnp.int32),
        pltpu.VMEM((2, K, SEQ_LEN, EMBED_DIM), jnp.float32),
        pltpu.SemaphoreType.DMA,
        pltpu.SemaphoreType.DMA,
        pltpu.SemaphoreType.DMA,
    ],
)
def _embed_lookup(idx_hbm, table_hbm, out_hbm, idx_v, rows_v, gsem,
                  ssem0, ssem1):
    ssem = (ssem0, ssem1)
    wid = lax.axis_index("s") * NC + lax.axis_index("c")
    b0 = wid * B_PER_W
    pltpu.sync_copy(idx_hbm.at[pl.ds(b0, B_PER_W)], idx_v)

    def halves(g, b):
        bb = g * K + b
        pieces = []
        for s0, n in ((0, SPLIT), (SPLIT, SEQ_LEN - SPLIT)):
            idx = idx_v.at[bb, pl.ds(s0, n)]
            pieces.append((idx, pl.ds(s0, n)))
        return bb, pieces

    def gather_fire(g, h):
        for b in range(K):
            _, pieces = halves(g, b)
            for idx, dst in pieces:
                pltpu.async_copy(table_hbm.at[idx], rows_v.at[h, b, dst],
                                 gsem)

    def gather_drain(g, h):
        for b in range(K):
            _, pieces = halves(g, b)
            for idx, dst in pieces:
                pltpu.make_async_copy(table_hbm.at[idx],
                                      rows_v.at[h, b, dst], gsem).wait()

    def store_fire(g, h):
        for b in range(K):
            bb, _ = halves(g, b)
            pltpu.async_copy(rows_v.at[h, b], out_hbm.at[b0 + bb], ssem[h])

    def store_drain(g, h):
        for b in range(K):
            bb, _ = halves(g, b)
            pltpu.make_async_copy(rows_v.at[h, b], out_hbm.at[b0 + bb],
                                  ssem[h]).wait()

    for h in range(2):
        gather_fire(h, h)
        gather_drain(h, h)
        store_fire(h, h)

    def group_pair(p, carry):
        for h in range(2):
            g = 2 * p + h
            store_drain(g - 2, h)
            gather_fire(g, h)
            gather_drain(g, h)
            store_fire(g, h)
        return carry

    lax.fori_loop(1, GROUPS // 2, group_pair, 0)

    store_drain(GROUPS - 2, 0)
    store_drain(GROUPS - 1, 1)


def kernel(x, word2vec_matrix):
    return _embed_lookup(x, word2vec_matrix)


# speedup vs baseline: 4.3764x; 2.2799x over previous
import functools

import jax
import jax.numpy as jnp
from jax import lax
from jax.experimental import pallas as pl
from jax.experimental.pallas import tpu as pltpu
from jax.experimental.pallas import tpu_sc as plsc

VOCAB = 100000
EMBED_DIM = 64
BATCH = 4096
SEQ_LEN = 200

NC = 2
NS = 16
NW = NC * NS

BBLK = BATCH // NW
DT = EMBED_DIM // 8
K = 2
GROUPS = SEQ_LEN // K

_GATHER_DN = lax.GatherDimensionNumbers(
    offset_dims=(), collapsed_slice_dims=(0,), start_index_map=(0,))


def _perm(v, idx2d):
    return lax.gather(v, idx2d, _GATHER_DN, (1,),
                      mode=lax.GatherScatterMode.PROMISE_IN_BOUNDS)


@functools.partial(
    pl.kernel,
    out_type=jax.ShapeDtypeStruct((SEQ_LEN, DT, NW, 8, 128), jnp.float32),
    mesh=plsc.VectorSubcoreMesh(core_axis_name="c", subcore_axis_name="s"),
    compiler_params=pltpu.CompilerParams(use_tc_tiling_on_sc=False,
                                         needs_layout_passes=False),
    scratch_types=[
        pltpu.VMEM((SEQ_LEN, BBLK), jnp.int32),
        pltpu.VMEM((2, K, BBLK, EMBED_DIM), jnp.float32),
        pltpu.VMEM((2, K, DT, 8, 128), jnp.float32),
        pltpu.SemaphoreType.DMA,
        pltpu.SemaphoreType.DMA,
        pltpu.SemaphoreType.DMA,
    ],
)
def _embed_lookup(idx_hbm, table_hbm, out_hbm, idx_v, rows_v, tile_v, gsem,
                  ssem0, ssem1):
    ssem = (ssem0, ssem1)
    wid = lax.axis_index("s") * NC + lax.axis_index("c")
    pltpu.sync_copy(idx_hbm.at[:, pl.ds(wid * BBLK, BBLK)], idx_v)

    def gather_fire(g, h):
        for b in range(K):
            pltpu.async_copy(table_hbm.at[idx_v.at[g * K + b]],
                             rows_v.at[h, b], gsem)

    def gather_drain(g, h):
        for b in range(K):
            pltpu.make_async_copy(table_hbm.at[idx_v.at[g * K + b]],
                                  rows_v.at[h, b], gsem).wait()

    def store_fire(g, h):
        for b in range(K):
            pltpu.async_copy(tile_v.at[h, b],
                             out_hbm.at[g * K + b, slice(None), wid],
                             ssem[h])

    def store_drain(g, h):
        for b in range(K):
            pltpu.make_async_copy(tile_v.at[h, b],
                                  out_hbm.at[g * K + b, slice(None), wid],
                                  ssem[h]).wait()

    lanes = lax.iota(jnp.int32, 16)
    rot_m = {m: lax.rem(lanes + (16 - m), 16)[:, None] for m in (8, 4, 2, 1)}
    rot_p = {m: lax.rem(lanes + m, 16)[:, None] for m in (8, 4, 2, 1)}
    masks = {m: (lanes & m) == 0 for m in (8, 4, 2, 1)}

    def transpose(h):
        for b in range(K):
            src = rows_v.at[h, b]
            dst = tile_v.at[h, b]

            def tc(c16, carry):
                base = c16 * 16
                for j in range(4):
                    a = [src[base + i, pl.ds(16 * j, 16)] for i in range(16)]
                    for m in (8, 4, 2, 1):
                        na = list(a)
                        for i in range(16):
                            if i & m:
                                continue
                            x, y = a[i], a[i | m]
                            na[i] = jnp.where(masks[m], x, _perm(y, rot_m[m]))
                            na[i | m] = jnp.where(masks[m],
                                                  _perm(x, rot_p[m]), y)
                        a = na
                    for i in range(16):
                        d = 16 * j + i
                        dst[d // 8, d % 8, pl.ds(base, 16)] = a[i]
                return carry

            lax.fori_loop(0, BBLK // 16, tc, 0)

    gather_fire(0, 0)

    def group_pair(p, carry):
        for h in range(2):
            g = 2 * p + h
            gather_drain(g, h)
            gather_fire(g + 1, 1 - h)
            store_drain(g - 2, h)
            transpose(h)
            store_fire(g, h)
        return carry

    for h in range(2):
        g = h
        gather_drain(g, h)
        gather_fire(g + 1, 1 - h)
        transpose(h)
        store_fire(g, h)

    lax.fori_loop(1, GROUPS // 2 - 1, group_pair, 0)

    for h in range(2):
        g = GROUPS - 2 + h
        gather_drain(g, h)
        if g + 1 < GROUPS:
            gather_fire(g + 1, 1 - h)
        store_drain(g - 2, h)
        transpose(h)
        store_fire(g, h)
    store_drain(GROUPS - 2, 0)
    store_drain(GROUPS - 1, 1)


def kernel(x, word2vec_matrix):
    o = _embed_lookup(x.T, word2vec_matrix)
    return o.transpose(2, 4, 0, 1, 3).reshape(BATCH, SEQ_LEN, EMBED_DIM)


# speedup vs baseline: 4.3765x; 1.0000x over previous
import functools

import jax
import jax.numpy as jnp
from jax import lax
from jax.experimental import pallas as pl
from jax.experimental.pallas import tpu as pltpu
from jax.experimental.pallas import tpu_sc as plsc

VOCAB = 100000
EMBED_DIM = 64
BATCH = 4096
SEQ_LEN = 200

NC = 2
NS = 16
NW = NC * NS

BBLK = BATCH // NW
DT = EMBED_DIM // 8
K = 2
GROUPS = SEQ_LEN // K

_GATHER_DN = lax.GatherDimensionNumbers(
    offset_dims=(), collapsed_slice_dims=(0,), start_index_map=(0,))


def _perm(v, idx2d):
    return lax.gather(v, idx2d, _GATHER_DN, (1,),
                      mode=lax.GatherScatterMode.PROMISE_IN_BOUNDS)


@functools.partial(
    pl.kernel,
    out_type=jax.ShapeDtypeStruct((SEQ_LEN, DT, NW, 8, 128), jnp.float32),
    mesh=plsc.VectorSubcoreMesh(core_axis_name="c", subcore_axis_name="s"),
    compiler_params=pltpu.CompilerParams(use_tc_tiling_on_sc=False,
                                         needs_layout_passes=False),
    scratch_types=[
        pltpu.VMEM((SEQ_LEN // 8, 8, BBLK), jnp.int32),
        pltpu.VMEM((2, K, BBLK, EMBED_DIM), jnp.float32),
        pltpu.VMEM((2, K, DT, 8, 128), jnp.float32),
        pltpu.SemaphoreType.DMA,
        pltpu.SemaphoreType.DMA,
        pltpu.SemaphoreType.DMA,
    ],
)
def _embed_lookup(idx_hbm, table_hbm, out_hbm, idx_v, rows_v, tile_v, gsem,
                  ssem0, ssem1):
    ssem = (ssem0, ssem1)
    wid = lax.axis_index("s") * NC + lax.axis_index("c")
    pltpu.sync_copy(idx_hbm.at[:, wid], idx_v)

    def idx_row(s):
        return idx_v.at[lax.div(s, 8), lax.rem(s, 8)]

    def gather_fire(g, h):
        for b in range(K):
            pltpu.async_copy(table_hbm.at[idx_row(g * K + b)],
                             rows_v.at[h, b], gsem)

    def gather_drain(g, h):
        for b in range(K):
            pltpu.make_async_copy(table_hbm.at[idx_row(g * K + b)],
                                  rows_v.at[h, b], gsem).wait()

    def store_fire(g, h):
        for b in range(K):
            pltpu.async_copy(tile_v.at[h, b],
                             out_hbm.at[g * K + b, slice(None), wid],
                             ssem[h])

    def store_drain(g, h):
        for b in range(K):
            pltpu.make_async_copy(tile_v.at[h, b],
                                  out_hbm.at[g * K + b, slice(None), wid],
                                  ssem[h]).wait()

    lanes = lax.iota(jnp.int32, 16)
    rot_m = {m: lax.rem(lanes + (16 - m), 16)[:, None] for m in (8, 4, 2, 1)}
    rot_p = {m: lax.rem(lanes + m, 16)[:, None] for m in (8, 4, 2, 1)}
    masks = {m: (lanes & m) == 0 for m in (8, 4, 2, 1)}

    def transpose(h):
        for b in range(K):
            src = rows_v.at[h, b]
            dst = tile_v.at[h, b]

            def tc(c16, carry):
                base = c16 * 16
                for j in range(4):
                    a = [src[base + i, pl.ds(16 * j, 16)] for i in range(16)]
                    for m in (8, 4, 2, 1):
                        na = list(a)
                        for i in range(16):
                            if i & m:
                                continue
                            x, y = a[i], a[i | m]
                            na[i] = jnp.where(masks[m], x, _perm(y, rot_m[m]))
                            na[i | m] = jnp.where(masks[m],
                                                  _perm(x, rot_p[m]), y)
                        a = na
                    for i in range(16):
                        d = 16 * j + i
                        dst[d // 8, d % 8, pl.ds(base, 16)] = a[i]
                return carry

            lax.fori_loop(0, BBLK // 16, tc, 0)

    gather_fire(0, 0)

    def group_pair(p, carry):
        for h in range(2):
            g = 2 * p + h
            gather_drain(g, h)
            gather_fire(g + 1, 1 - h)
            store_drain(g - 2, h)
            transpose(h)
            store_fire(g, h)
        return carry

    for h in range(2):
        g = h
        gather_drain(g, h)
        gather_fire(g + 1, 1 - h)
        transpose(h)
        store_fire(g, h)

    lax.fori_loop(1, GROUPS // 2 - 1, group_pair, 0)

    for h in range(2):
        g = GROUPS - 2 + h
        gather_drain(g, h)
        if g + 1 < GROUPS:
            gather_fire(g + 1, 1 - h)
        store_drain(g - 2, h)
        transpose(h)
        store_fire(g, h)
    store_drain(GROUPS - 2, 0)
    store_drain(GROUPS - 1, 1)


def kernel(x, word2vec_matrix):
    x5 = x.reshape(NW, BBLK, SEQ_LEN // 8, 8).transpose(2, 0, 3, 1)
    o = _embed_lookup(x5, word2vec_matrix)
    return o.transpose(2, 4, 0, 1, 3).reshape(BATCH, SEQ_LEN, EMBED_DIM)
